# baseline (speedup 1.0000x reference)
import functools
import math

import jax
import jax.numpy as jnp
from jax import lax
from jax.experimental import pallas as pl
from jax.experimental.pallas import tpu as pltpu

N = 10000
E = 320000
D = 128
L = 3


_ERFC_P = (+2.326819970068386e-2, -1.387039388740657e-1, +3.687424674597105e-1,
           -5.824733027278666e-1, +6.210004621745983e-1, -4.944515323274145e-1,
           +3.404879937665872e-1, -2.741127028184656e-1, +5.638259427386472e-1)
_ERFC_R = (-1.047766399936249e+1, +1.297719955372516e+1, -7.495518717768503e+0,
           +2.921019019210786e+0, -1.015265279202700e+0, +4.218463358204948e-1,
           -2.820767439740514e-1, +5.641895067754075e-1)
_ERF_T = (+7.853861353153693e-5, -8.010193625184903e-4, +5.188327685732524e-3,
          -2.685381193529856e-2, +1.128358514861418e-1, -3.761262582423300e-1,
          +1.128379165726710e+0)


def _poly(y, coeffs):
    p = jnp.full_like(y, jnp.float32(coeffs[0]))
    for c in coeffs[1:]:
        p = p * y + jnp.float32(c)
    return p


def _erfc(x):
    abs_x = jnp.abs(x)
    z = jnp.exp(-x * x)
    q = jnp.float32(1.0) / abs_x
    y = q * q
    p = jnp.where(abs_x < 2.0, _poly(y, _ERFC_P), _poly(y, _ERFC_R))
    yv = z * q * p
    y_clamp = jnp.where(-x * x < jnp.float32(-88.72283905206835),
                        jnp.float32(0.0), yv)
    big = jnp.where(x < 0.0, jnp.float32(2.0) - y_clamp, y_clamp)
    small = jnp.float32(1.0) - x * _poly(x * x, _ERF_T)
    return jnp.where(abs_x > 1.0, big, small)


def _gelu(x):
    return 0.5 * x * _erfc(-x * math.sqrt(0.5))


def _bn(t, g, b):
    m = jnp.mean(t, axis=0, keepdims=True)
    v = jnp.mean((t - m) ** 2, axis=0, keepdims=True)
    return (t - m) / jnp.sqrt(v + 1e-5) * g + b


def _dot(a, b):
    return jnp.dot(a, b, preferred_element_type=jnp.float32)


def _stem_body(x_ref, w0_ref, sw1_ref, sb1_ref, sg1_ref, sB1_ref,
               sw2_ref, sb2_ref, sg2_ref, sB2_ref, out_ref):
    h = _dot(x_ref[...], w0_ref[...])
    t = _gelu(_dot(h, sw1_ref[...]) + sb1_ref[...])
    t = _bn(t, sg1_ref[...], sB1_ref[...])
    t = _gelu(_dot(t, sw2_ref[...]) + sb2_ref[...])
    out_ref[...] = _bn(t, sg2_ref[...], sB2_ref[...]) + t


def _stem(x, W0, sW1, sb1, sg1, sB1, sW2, sb2, sg2, sB2):
    return pl.pallas_call(
        _stem_body,
        out_shape=jax.ShapeDtypeStruct((N, D), jnp.float32),
    )(x, W0, sW1.reshape(D, D), sb1.reshape(1, D), sg1.reshape(1, D),
      sB1.reshape(1, D), sW2.reshape(D, D), sb2.reshape(1, D),
      sg2.reshape(1, D), sB2.reshape(1, D))


def _pre_body(f_ref, w_ref, rw_ref, rb_ref, msgs_ref, rpart_ref):
    msgs_ref[...] = _dot(f_ref[...], w_ref[...])
    rpart_ref[...] = jax.nn.relu(_dot(f_ref[...], rw_ref[...]) + rb_ref[...])


def _pre(f, w, rw, rb):
    return pl.pallas_call(
        _pre_body,
        out_shape=(jax.ShapeDtypeStruct((N, D), jnp.float32),
                   jax.ShapeDtypeStruct((N, D), jnp.float32)),
    )(f, w, rw, rb.reshape(1, D))


def _postz_body(agg_ref, rpart_ref, gb_ref, out_ref):
    out_ref[...] = jax.nn.relu(agg_ref[0] + agg_ref[1] + gb_ref[...]) \
        + rpart_ref[...]


def _postz(agg2, rpart, gb):
    return pl.pallas_call(
        _postz_body,
        out_shape=jax.ShapeDtypeStruct((N, D), jnp.float32),
    )(agg2, rpart, gb.reshape(1, D))


def _posts_body(agg_ref, rpart_ref, mb_ref, mg_ref, mB_ref, out_ref):
    s2 = jax.nn.relu(agg_ref[0] + agg_ref[1] + mb_ref[...]) + rpart_ref[...]
    out_ref[...] = _bn(s2, mg_ref[...], mB_ref[...])


def _posts(agg2, rpart, mb, mg, mB):
    return pl.pallas_call(
        _posts_body,
        out_shape=jax.ShapeDtypeStruct((N, D), jnp.float32),
    )(agg2, rpart, mb.reshape(1, D), mg.reshape(1, D), mB.reshape(1, D))


def _seg_sum(table, src, dst):
    agg = jax.ops.segment_sum(table[src], dst, num_segments=N)
    return jnp.stack([agg, jnp.zeros_like(agg)], 0)


def _seg_max(table, src, dst):
    p = jax.ops.segment_max(table[src], dst, num_segments=N)
    return jnp.where(jnp.isfinite(p), p, 0.0)


def kernel(x, edge_index, W0, sW1, sb1, sg1, sB1, sW2, sb2, sg2, sB2,
           gW, gb, rW, rb, mW, mb, mrW, mrb, mg, mB):
    src = edge_index[0]
    dst = edge_index[1]
    x_p = jnp.pad(x, ((0, 0), (0, D - x.shape[1])))
    W0_p = jnp.pad(W0, ((0, D - W0.shape[0]), (0, 0)))
    feats = _stem(x_p, W0_p, sW1, sb1, sg1, sB1, sW2, sb2, sg2, sB2)
    outs = []
    for i in range(L):
        msgs_g, rpart = _pre(feats, gW[i], rW[i], rb[i])
        aggG = _seg_sum(msgs_g, src, dst)
        z = _postz(aggG, rpart, gb[i])
        p = _seg_max(z, src, dst)
        msgs_m, rpart2 = _pre(p, mW[i], mrW[i], mrb[i])
        aggM = _seg_sum(msgs_m, src, dst)
        outs.append(_posts(aggM, rpart2, mb[i], mg[i], mB[i]))
        feats = z
    return jnp.stack(outs, 0)


# speedup vs baseline: 2.0200x; 2.0200x over previous
import functools
import math

import jax
import jax.numpy as jnp
from jax import lax
from jax.experimental import pallas as pl
from jax.experimental.pallas import tpu as pltpu
from jax.experimental.pallas import tpu_sc as plsc

N = 10000
E = 320000
D = 128
L = 3

_NC = 2
_NS = 16
_NW = _NC * _NS
_EPW = E // _NW
_CH = 125
_NCHUNK = _EPW // _CH
_RPS = 640
_NPAD = _RPS * _NS


_ERFC_P = (+2.326819970068386e-2, -1.387039388740657e-1, +3.687424674597105e-1,
           -5.824733027278666e-1, +6.210004621745983e-1, -4.944515323274145e-1,
           +3.404879937665872e-1, -2.741127028184656e-1, +5.638259427386472e-1)
_ERFC_R = (-1.047766399936249e+1, +1.297719955372516e+1, -7.495518717768503e+0,
           +2.921019019210786e+0, -1.015265279202700e+0, +4.218463358204948e-1,
           -2.820767439740514e-1, +5.641895067754075e-1)
_ERF_T = (+7.853861353153693e-5, -8.010193625184903e-4, +5.188327685732524e-3,
          -2.685381193529856e-2, +1.128358514861418e-1, -3.761262582423300e-1,
          +1.128379165726710e+0)


def _poly(y, coeffs):
    p = jnp.full_like(y, jnp.float32(coeffs[0]))
    for c in coeffs[1:]:
        p = p * y + jnp.float32(c)
    return p


def _erfc(x):
    abs_x = jnp.abs(x)
    z = jnp.exp(-x * x)
    q = jnp.float32(1.0) / abs_x
    y = q * q
    p = jnp.where(abs_x < 2.0, _poly(y, _ERFC_P), _poly(y, _ERFC_R))
    yv = z * q * p
    y_clamp = jnp.where(-x * x < jnp.float32(-88.72283905206835),
                        jnp.float32(0.0), yv)
    big = jnp.where(x < 0.0, jnp.float32(2.0) - y_clamp, y_clamp)
    small = jnp.float32(1.0) - x * _poly(x * x, _ERF_T)
    return jnp.where(abs_x > 1.0, big, small)


def _gelu(x):
    return 0.5 * x * _erfc(-x * math.sqrt(0.5))


def _bn(t, g, b):
    m = jnp.mean(t, axis=0, keepdims=True)
    v = jnp.mean((t - m) ** 2, axis=0, keepdims=True)
    return (t - m) / jnp.sqrt(v + 1e-5) * g + b


def _dot(a, b):
    return jnp.dot(a, b, preferred_element_type=jnp.float32)


def _stem_body(x_ref, w0_ref, sw1_ref, sb1_ref, sg1_ref, sB1_ref,
               sw2_ref, sb2_ref, sg2_ref, sB2_ref, out_ref):
    h = _dot(x_ref[...], w0_ref[...])
    t = _gelu(_dot(h, sw1_ref[...]) + sb1_ref[...])
    t = _bn(t, sg1_ref[...], sB1_ref[...])
    t = _gelu(_dot(t, sw2_ref[...]) + sb2_ref[...])
    out_ref[...] = _bn(t, sg2_ref[...], sB2_ref[...]) + t


def _stem(x, W0, sW1, sb1, sg1, sB1, sW2, sb2, sg2, sB2):
    return pl.pallas_call(
        _stem_body,
        out_shape=jax.ShapeDtypeStruct((N, D), jnp.float32),
    )(x, W0, sW1.reshape(D, D), sb1.reshape(1, D), sg1.reshape(1, D),
      sB1.reshape(1, D), sW2.reshape(D, D), sb2.reshape(1, D),
      sg2.reshape(1, D), sB2.reshape(1, D))


def _pre_body(f_ref, w_ref, rw_ref, rb_ref, msgs_ref, rpart_ref):
    msgs_ref[...] = _dot(f_ref[...], w_ref[...])
    rpart_ref[...] = jax.nn.relu(_dot(f_ref[...], rw_ref[...]) + rb_ref[...])


def _pre(f, w, rw, rb):
    return pl.pallas_call(
        _pre_body,
        out_shape=(jax.ShapeDtypeStruct((N, D), jnp.float32),
                   jax.ShapeDtypeStruct((N, D), jnp.float32)),
    )(f, w, rw, rb.reshape(1, D))


def _postz_body(agg_ref, rpart_ref, gb_ref, out_ref):
    out_ref[...] = jax.nn.relu(agg_ref[0] + agg_ref[1] + gb_ref[...]) \
        + rpart_ref[...]


def _postz(agg2, rpart, gb):
    return pl.pallas_call(
        _postz_body,
        out_shape=jax.ShapeDtypeStruct((N, D), jnp.float32),
    )(agg2, rpart, gb.reshape(1, D))


def _posts_body(agg_ref, rpart_ref, mb_ref, mg_ref, mB_ref, out_ref):
    s2 = jax.nn.relu(agg_ref[0] + agg_ref[1] + mb_ref[...]) + rpart_ref[...]
    out_ref[...] = _bn(s2, mg_ref[...], mB_ref[...])


def _posts(agg2, rpart, mb, mg, mB):
    return pl.pallas_call(
        _posts_body,
        out_shape=jax.ShapeDtypeStruct((N, D), jnp.float32),
    )(agg2, rpart, mb.reshape(1, D), mg.reshape(1, D), mB.reshape(1, D))


def _segsum_body(table, src_r, dst_r, zeros, out, src_v, dst_v, gsem,
                 rows_v, acc_sh):
    c = lax.axis_index("c")
    s = lax.axis_index("s")
    wid = c * _NS + s
    if True:
        pltpu.sync_copy(zeros, acc_sh.at[pl.ds(s * _RPS, _RPS)])
        pltpu.sync_copy(src_r.at[wid], src_v)
        pltpu.sync_copy(dst_r.at[wid], dst_v)
        plsc.subcore_barrier()

        def step(j, carry):
            pltpu.async_copy(table.at[src_v.at[j]], rows_v, gsem).wait()
            pltpu.sync_copy(rows_v, acc_sh.at[dst_v.at[j]], add=True)
            return carry

        lax.fori_loop(0, _NCHUNK, step, 0)
        plsc.subcore_barrier()
        @pl.when(s < _NS - 1)
        def _():
            pltpu.sync_copy(acc_sh.at[pl.ds(s * _RPS, _RPS)],
                            out.at[c, pl.ds(s * _RPS, _RPS)])

        @pl.when(s == _NS - 1)
        def _():
            tail = N - (_NS - 1) * _RPS
            pltpu.sync_copy(acc_sh.at[pl.ds((_NS - 1) * _RPS, tail)],
                            out.at[c, pl.ds((_NS - 1) * _RPS, tail)])

_segsum_call = pl.kernel(
    _segsum_body,
    out_type=jax.ShapeDtypeStruct((_NC, N, D), jnp.float32),
    mesh=plsc.VectorSubcoreMesh(core_axis_name="c", subcore_axis_name="s"),
    scratch_types=[
        pltpu.VMEM((_NCHUNK, _CH), jnp.int32),
        pltpu.VMEM((_NCHUNK, _CH), jnp.int32),
        pltpu.SemaphoreType.DMA,
        pltpu.VMEM((_CH, D), jnp.float32),
        pltpu.VMEM_SHARED((_NPAD, D), jnp.float32),
    ],
)


def _seg_sum(table, src_r, dst_r, zeros):
    return _segsum_call(table, src_r, dst_r, zeros)


def _seg_max(table, src, dst):
    p = jax.ops.segment_max(table[src], dst, num_segments=N)
    return jnp.where(jnp.isfinite(p), p, 0.0)


def kernel(x, edge_index, W0, sW1, sb1, sg1, sB1, sW2, sb2, sg2, sB2,
           gW, gb, rW, rb, mW, mb, mrW, mrb, mg, mB):
    src = edge_index[0]
    dst = edge_index[1]
    src_r = src.reshape(_NW, _NCHUNK, _CH)
    dst_r = dst.reshape(_NW, _NCHUNK, _CH)
    zeros = jnp.zeros((_RPS, D), jnp.float32)
    x_p = jnp.pad(x, ((0, 0), (0, D - x.shape[1])))
    W0_p = jnp.pad(W0, ((0, D - W0.shape[0]), (0, 0)))
    feats = _stem(x_p, W0_p, sW1, sb1, sg1, sB1, sW2, sb2, sg2, sB2)
    outs = []
    for i in range(L):
        msgs_g, rpart = _pre(feats, gW[i], rW[i], rb[i])
        aggG = _seg_sum(msgs_g, src_r, dst_r, zeros)
        z = _postz(aggG, rpart, gb[i])
        p = _seg_max(z, src, dst)
        msgs_m, rpart2 = _pre(p, mW[i], mrW[i], mrb[i])
        aggM = _seg_sum(msgs_m, src_r, dst_r, zeros)
        outs.append(_posts(aggM, rpart2, mb[i], mg[i], mB[i]))
        feats = z
    return jnp.stack(outs, 0)
